# baseline (speedup 1.0000x reference)
import jax
import jax.numpy as jnp
import numpy as np
from jax import lax
from jax.experimental import pallas as pl
from jax.experimental.pallas import tpu as pltpu
from jax.experimental.pallas import tpu_sc as plsc

B, E, C, HW, K = 32, 64, 768, 576, 8
NEG = -3.0e38
SCALE = float(np.float32(np.float32(1.0) / np.float32(HW)) / np.float32(128.0))
NC, NS, L = 2, 16, 16


def _pool_body(x_ref, w_ref, b_ref, out_ref, p_ref):
    bidx = pl.program_id(0)
    xb = x_ref[0].astype(jnp.bfloat16)
    logits = lax.dot_general(
        w_ref[...], xb, (((1,), (0,)), ((), ())),
        preferred_element_type=jnp.float32)
    ones = jnp.ones((HW, 128), jnp.float32)
    ssum = lax.dot_general(
        logits, ones, (((1,), (0,)), ((), ())),
        precision=lax.Precision.HIGHEST,
        preferred_element_type=jnp.float32)
    p_ref[pl.ds(bidx, 1)] = ssum[None]

    @pl.when(bidx == B - 1)
    def _finish():
        out_ref[...] = jnp.sum(p_ref[...], axis=2) * SCALE + b_ref[...]


def _lane_reduce(x, op):
    iota = lax.broadcasted_iota(jnp.int32, (L,), 0)
    dnums = lax.GatherDimensionNumbers(
        offset_dims=(), collapsed_slice_dims=(0,), start_index_map=(0,))
    for s in (8, 4, 2, 1):
        perm = jnp.reshape(iota ^ s, (L, 1))
        x = op(x, lax.gather(x, perm, dnums, (1,),
                             mode=lax.GatherScatterMode.PROMISE_IN_BOUNDS))
    return x


def _route_body(pooled_hbm, out_hbm, row_v, orow_v):
    wid = lax.axis_index("s") * NC + lax.axis_index("c")
    pltpu.sync_copy(pooled_hbm.at[wid], row_v)
    iota = lax.broadcasted_iota(jnp.int32, (L,), 0)
    v = [row_v[pl.ds(L * j, L)] for j in range(E // L)]
    gidx = [iota + L * j for j in range(E // L)]

    sel_vals = jnp.zeros((L,), jnp.float32)
    sel_idx = jnp.zeros((L,), jnp.int32)
    row_max = None
    for k in range(K):
        m = _lane_reduce(jnp.maximum(jnp.maximum(v[0], v[1]),
                                     jnp.maximum(v[2], v[3])), jnp.maximum)
        cand = [jnp.where(v[j] == m, gidx[j], E) for j in range(E // L)]
        idx = _lane_reduce(jnp.minimum(jnp.minimum(cand[0], cand[1]),
                                       jnp.minimum(cand[2], cand[3])),
                           jnp.minimum)
        sel_vals = jnp.where(iota == k, m, sel_vals)
        sel_idx = jnp.where(iota == k, idx, sel_idx)
        if row_max is None:
            row_max = m
        v = [jnp.where(gidx[j] == idx, jnp.float32(NEG), v[j])
             for j in range(E // L)]

    kmask = iota < K
    e = jnp.where(kmask, jnp.exp(sel_vals - row_max), 0.0)
    probs = e / _lane_reduce(e, jnp.add)

    zeros = jnp.zeros((L,), jnp.float32)
    for j in range(E // L):
        orow_v[pl.ds(L * j, L)] = zeros
    plsc.store_scatter(orow_v, [sel_idx], probs, mask=kmask)
    pltpu.sync_copy(orow_v, out_hbm.at[wid])


def kernel(x, W, b):
    xr = x.reshape(B, C, HW)
    Wb = W.astype(jnp.bfloat16)
    pooled = pl.pallas_call(
        _pool_body,
        grid=(B,),
        in_specs=[
            pl.BlockSpec((1, C, HW), lambda i: (i, 0, 0)),
            pl.BlockSpec((E, C), lambda i: (0, 0)),
            pl.BlockSpec((1, E), lambda i: (0, 0)),
        ],
        out_specs=pl.BlockSpec((B, E), lambda i: (0, 0)),
        out_shape=jax.ShapeDtypeStruct((B, E), jnp.float32),
        scratch_shapes=[pltpu.VMEM((B, E, 128), jnp.float32)],
        compiler_params=pltpu.CompilerParams(
            dimension_semantics=("arbitrary",)),
    )(xr, Wb, b.reshape(1, E))

    route = pl.kernel(
        _route_body,
        mesh=plsc.VectorSubcoreMesh(core_axis_name="c", subcore_axis_name="s"),
        out_type=jax.ShapeDtypeStruct((B, E), jnp.float32),
        scratch_types=[pltpu.VMEM((E,), jnp.float32),
                       pltpu.VMEM((E,), jnp.float32)],
        compiler_params=pltpu.CompilerParams(needs_layout_passes=False),
    )
    return route(pooled)


# speedup vs baseline: 1.0946x; 1.0946x over previous
import jax
import jax.numpy as jnp
import numpy as np
from jax import lax
from jax.experimental import pallas as pl
from jax.experimental.pallas import tpu as pltpu
from jax.experimental.pallas import tpu_sc as plsc

B, E, C, HW, K = 32, 64, 768, 576, 8
NEG = -3.0e38
SCALE = float(np.float32(np.float32(1.0) / np.float32(HW)) / np.float32(128.0))
NC, NS, L = 2, 16, 16


BPB = 2


def _pool_body(x_ref, w_ref, b_ref, out_ref, p_ref):
    bidx = pl.program_id(0)
    ones = jnp.ones((HW, 128), jnp.float32)
    for j in range(BPB):
        xb = x_ref[j].astype(jnp.bfloat16)
        logits = lax.dot_general(
            w_ref[...], xb, (((1,), (0,)), ((), ())),
            preferred_element_type=jnp.float32)
        ssum = lax.dot_general(
            logits, ones, (((1,), (0,)), ((), ())),
            precision=lax.Precision.HIGHEST,
            preferred_element_type=jnp.float32)
        p_ref[pl.ds(bidx * BPB + j, 1)] = ssum[None]

    @pl.when(bidx == B // BPB - 1)
    def _finish():
        out_ref[...] = jnp.sum(p_ref[...], axis=2) * SCALE + b_ref[...]


def _lane_reduce(x, op):
    iota = lax.broadcasted_iota(jnp.int32, (L,), 0)
    dnums = lax.GatherDimensionNumbers(
        offset_dims=(), collapsed_slice_dims=(0,), start_index_map=(0,))
    for s in (8, 4, 2, 1):
        perm = jnp.reshape(iota ^ s, (L, 1))
        x = op(x, lax.gather(x, perm, dnums, (1,),
                             mode=lax.GatherScatterMode.PROMISE_IN_BOUNDS))
    return x


def _route_body(pooled_hbm, out_hbm, row_v, orow_v):
    wid = lax.axis_index("s") * NC + lax.axis_index("c")
    pltpu.sync_copy(pooled_hbm.at[wid], row_v)
    iota = lax.broadcasted_iota(jnp.int32, (L,), 0)
    v = [row_v[pl.ds(L * j, L)] for j in range(E // L)]
    gidx = [iota + L * j for j in range(E // L)]

    sel_vals = jnp.zeros((L,), jnp.float32)
    sel_idx = jnp.zeros((L,), jnp.int32)
    row_max = None
    for k in range(K):
        m = _lane_reduce(jnp.maximum(jnp.maximum(v[0], v[1]),
                                     jnp.maximum(v[2], v[3])), jnp.maximum)
        cand = [jnp.where(v[j] == m, gidx[j], E) for j in range(E // L)]
        idx = _lane_reduce(jnp.minimum(jnp.minimum(cand[0], cand[1]),
                                       jnp.minimum(cand[2], cand[3])),
                           jnp.minimum)
        sel_vals = jnp.where(iota == k, m, sel_vals)
        sel_idx = jnp.where(iota == k, idx, sel_idx)
        if row_max is None:
            row_max = m
        v = [jnp.where(gidx[j] == idx, jnp.float32(NEG), v[j])
             for j in range(E // L)]

    kmask = iota < K
    e = jnp.where(kmask, jnp.exp(sel_vals - row_max), 0.0)
    probs = e / _lane_reduce(e, jnp.add)

    zeros = jnp.zeros((L,), jnp.float32)
    for j in range(E // L):
        orow_v[pl.ds(L * j, L)] = zeros
    plsc.store_scatter(orow_v, [sel_idx], probs, mask=kmask)
    pltpu.sync_copy(orow_v, out_hbm.at[wid])


def kernel(x, W, b):
    xr = x.reshape(B, C, HW)
    Wb = W.astype(jnp.bfloat16)
    pooled = pl.pallas_call(
        _pool_body,
        grid=(B // BPB,),
        in_specs=[
            pl.BlockSpec((BPB, C, HW), lambda i: (i, 0, 0)),
            pl.BlockSpec((E, C), lambda i: (0, 0)),
            pl.BlockSpec((1, E), lambda i: (0, 0)),
        ],
        out_specs=pl.BlockSpec((B, E), lambda i: (0, 0)),
        out_shape=jax.ShapeDtypeStruct((B, E), jnp.float32),
        scratch_shapes=[pltpu.VMEM((B, E, 128), jnp.float32)],
        compiler_params=pltpu.CompilerParams(
            dimension_semantics=("arbitrary",)),
    )(xr, Wb, b.reshape(1, E))

    route = pl.kernel(
        _route_body,
        mesh=plsc.VectorSubcoreMesh(core_axis_name="c", subcore_axis_name="s"),
        out_type=jax.ShapeDtypeStruct((B, E), jnp.float32),
        scratch_types=[pltpu.VMEM((E,), jnp.float32),
                       pltpu.VMEM((E,), jnp.float32)],
        compiler_params=pltpu.CompilerParams(needs_layout_passes=False),
    )
    return route(pooled)


# speedup vs baseline: 1.1355x; 1.0373x over previous
import jax
import jax.numpy as jnp
import numpy as np
from jax import lax
from jax.experimental import pallas as pl
from jax.experimental.pallas import tpu as pltpu
from jax.experimental.pallas import tpu_sc as plsc

B, E, C, HW, K = 32, 64, 768, 576, 8
NEG = -3.0e38
SCALE = float(np.float32(np.float32(1.0) / np.float32(HW)) / np.float32(128.0))
NC, NS, L = 2, 16, 16


BPB = 4


def _pool_body(x_ref, w_ref, b_ref, out_ref, p_ref):
    bidx = pl.program_id(0)
    ones = jnp.ones((HW, 128), jnp.float32)
    for j in range(BPB):
        xb = x_ref[j].astype(jnp.bfloat16)
        logits = lax.dot_general(
            w_ref[...], xb, (((1,), (0,)), ((), ())),
            preferred_element_type=jnp.float32)
        ssum = lax.dot_general(
            logits, ones, (((1,), (0,)), ((), ())),
            precision=lax.Precision.HIGHEST,
            preferred_element_type=jnp.float32)
        p_ref[pl.ds(bidx * BPB + j, 1)] = ssum[None]

    @pl.when(bidx == B // BPB - 1)
    def _finish():
        out_ref[...] = jnp.sum(p_ref[...], axis=2) * SCALE + b_ref[...]


def _lane_reduce(x, op):
    iota = lax.broadcasted_iota(jnp.int32, (L,), 0)
    dnums = lax.GatherDimensionNumbers(
        offset_dims=(), collapsed_slice_dims=(0,), start_index_map=(0,))
    for s in (8, 4, 2, 1):
        perm = jnp.reshape(iota ^ s, (L, 1))
        x = op(x, lax.gather(x, perm, dnums, (1,),
                             mode=lax.GatherScatterMode.PROMISE_IN_BOUNDS))
    return x


def _route_body(pooled_hbm, out_hbm, row_v, orow_v):
    wid = lax.axis_index("s") * NC + lax.axis_index("c")
    pltpu.sync_copy(pooled_hbm.at[wid], row_v)
    iota = lax.broadcasted_iota(jnp.int32, (L,), 0)
    v = [row_v[pl.ds(L * j, L)] for j in range(E // L)]
    gidx = [iota + L * j for j in range(E // L)]

    sel_vals = jnp.zeros((L,), jnp.float32)
    sel_idx = jnp.zeros((L,), jnp.int32)
    row_max = None
    for k in range(K):
        m = _lane_reduce(jnp.maximum(jnp.maximum(v[0], v[1]),
                                     jnp.maximum(v[2], v[3])), jnp.maximum)
        cand = [jnp.where(v[j] == m, gidx[j], E) for j in range(E // L)]
        idx = _lane_reduce(jnp.minimum(jnp.minimum(cand[0], cand[1]),
                                       jnp.minimum(cand[2], cand[3])),
                           jnp.minimum)
        sel_vals = jnp.where(iota == k, m, sel_vals)
        sel_idx = jnp.where(iota == k, idx, sel_idx)
        if row_max is None:
            row_max = m
        v = [jnp.where(gidx[j] == idx, jnp.float32(NEG), v[j])
             for j in range(E // L)]

    kmask = iota < K
    e = jnp.where(kmask, jnp.exp(sel_vals - row_max), 0.0)
    probs = e / _lane_reduce(e, jnp.add)

    zeros = jnp.zeros((L,), jnp.float32)
    for j in range(E // L):
        orow_v[pl.ds(L * j, L)] = zeros
    plsc.store_scatter(orow_v, [sel_idx], probs, mask=kmask)
    pltpu.sync_copy(orow_v, out_hbm.at[wid])


def kernel(x, W, b):
    xr = x.reshape(B, C, HW)
    Wb = W.astype(jnp.bfloat16)
    pooled = pl.pallas_call(
        _pool_body,
        grid=(B // BPB,),
        in_specs=[
            pl.BlockSpec((BPB, C, HW), lambda i: (i, 0, 0)),
            pl.BlockSpec((E, C), lambda i: (0, 0)),
            pl.BlockSpec((1, E), lambda i: (0, 0)),
        ],
        out_specs=pl.BlockSpec((B, E), lambda i: (0, 0)),
        out_shape=jax.ShapeDtypeStruct((B, E), jnp.float32),
        scratch_shapes=[pltpu.VMEM((B, E, 128), jnp.float32)],
        compiler_params=pltpu.CompilerParams(
            dimension_semantics=("arbitrary",)),
    )(xr, Wb, b.reshape(1, E))

    route = pl.kernel(
        _route_body,
        mesh=plsc.VectorSubcoreMesh(core_axis_name="c", subcore_axis_name="s"),
        out_type=jax.ShapeDtypeStruct((B, E), jnp.float32),
        scratch_types=[pltpu.VMEM((E,), jnp.float32),
                       pltpu.VMEM((E,), jnp.float32)],
        compiler_params=pltpu.CompilerParams(needs_layout_passes=False),
    )
    return route(pooled)


# speedup vs baseline: 1.1366x; 1.0010x over previous
import jax
import jax.numpy as jnp
import numpy as np
from jax import lax
from jax.experimental import pallas as pl
from jax.experimental.pallas import tpu as pltpu
from jax.experimental.pallas import tpu_sc as plsc

B, E, C, HW, K = 32, 64, 768, 576, 8
NEG = -3.0e38
SCALE = float(np.float32(np.float32(1.0) / np.float32(HW)) / np.float32(128.0))
NC, NS, L = 2, 16, 16


BPB = 8


def _pool_body(x_ref, w_ref, b_ref, out_ref, p_ref):
    bidx = pl.program_id(0)
    ones = jnp.ones((HW, 128), jnp.float32)
    for j in range(BPB):
        xb = x_ref[j].astype(jnp.bfloat16)
        logits = lax.dot_general(
            w_ref[...], xb, (((1,), (0,)), ((), ())),
            preferred_element_type=jnp.float32)
        ssum = lax.dot_general(
            logits, ones, (((1,), (0,)), ((), ())),
            precision=lax.Precision.HIGHEST,
            preferred_element_type=jnp.float32)
        p_ref[pl.ds(bidx * BPB + j, 1)] = ssum[None]

    @pl.when(bidx == B // BPB - 1)
    def _finish():
        out_ref[...] = jnp.sum(p_ref[...], axis=2) * SCALE + b_ref[...]


def _lane_reduce(x, op):
    iota = lax.broadcasted_iota(jnp.int32, (L,), 0)
    dnums = lax.GatherDimensionNumbers(
        offset_dims=(), collapsed_slice_dims=(0,), start_index_map=(0,))
    for s in (8, 4, 2, 1):
        perm = jnp.reshape(iota ^ s, (L, 1))
        x = op(x, lax.gather(x, perm, dnums, (1,),
                             mode=lax.GatherScatterMode.PROMISE_IN_BOUNDS))
    return x


def _route_body(pooled_hbm, out_hbm, row_v, orow_v):
    wid = lax.axis_index("s") * NC + lax.axis_index("c")
    pltpu.sync_copy(pooled_hbm.at[wid], row_v)
    iota = lax.broadcasted_iota(jnp.int32, (L,), 0)
    v = [row_v[pl.ds(L * j, L)] for j in range(E // L)]
    gidx = [iota + L * j for j in range(E // L)]

    sel_vals = jnp.zeros((L,), jnp.float32)
    sel_idx = jnp.zeros((L,), jnp.int32)
    row_max = None
    for k in range(K):
        m = _lane_reduce(jnp.maximum(jnp.maximum(v[0], v[1]),
                                     jnp.maximum(v[2], v[3])), jnp.maximum)
        cand = [jnp.where(v[j] == m, gidx[j], E) for j in range(E // L)]
        idx = _lane_reduce(jnp.minimum(jnp.minimum(cand[0], cand[1]),
                                       jnp.minimum(cand[2], cand[3])),
                           jnp.minimum)
        sel_vals = jnp.where(iota == k, m, sel_vals)
        sel_idx = jnp.where(iota == k, idx, sel_idx)
        if row_max is None:
            row_max = m
        v = [jnp.where(gidx[j] == idx, jnp.float32(NEG), v[j])
             for j in range(E // L)]

    kmask = iota < K
    e = jnp.where(kmask, jnp.exp(sel_vals - row_max), 0.0)
    probs = e / _lane_reduce(e, jnp.add)

    zeros = jnp.zeros((L,), jnp.float32)
    for j in range(E // L):
        orow_v[pl.ds(L * j, L)] = zeros
    plsc.store_scatter(orow_v, [sel_idx], probs, mask=kmask)
    pltpu.sync_copy(orow_v, out_hbm.at[wid])


def kernel(x, W, b):
    xr = x.reshape(B, C, HW)
    Wb = W.astype(jnp.bfloat16)
    pooled = pl.pallas_call(
        _pool_body,
        grid=(B // BPB,),
        in_specs=[
            pl.BlockSpec((BPB, C, HW), lambda i: (i, 0, 0)),
            pl.BlockSpec((E, C), lambda i: (0, 0)),
            pl.BlockSpec((1, E), lambda i: (0, 0)),
        ],
        out_specs=pl.BlockSpec((B, E), lambda i: (0, 0)),
        out_shape=jax.ShapeDtypeStruct((B, E), jnp.float32),
        scratch_shapes=[pltpu.VMEM((B, E, 128), jnp.float32)],
        compiler_params=pltpu.CompilerParams(
            dimension_semantics=("arbitrary",)),
    )(xr, Wb, b.reshape(1, E))

    route = pl.kernel(
        _route_body,
        mesh=plsc.VectorSubcoreMesh(core_axis_name="c", subcore_axis_name="s"),
        out_type=jax.ShapeDtypeStruct((B, E), jnp.float32),
        scratch_types=[pltpu.VMEM((E,), jnp.float32),
                       pltpu.VMEM((E,), jnp.float32)],
        compiler_params=pltpu.CompilerParams(needs_layout_passes=False),
    )
    return route(pooled)


# speedup vs baseline: 1.1407x; 1.0036x over previous
import jax
import jax.numpy as jnp
import numpy as np
from jax import lax
from jax.experimental import pallas as pl
from jax.experimental.pallas import tpu as pltpu
from jax.experimental.pallas import tpu_sc as plsc

B, E, C, HW, K = 32, 64, 768, 576, 8
NEG = -3.0e38
SCALE = float(np.float32(np.float32(1.0) / np.float32(HW)) / np.float32(128.0))
NC, NS, L = 2, 16, 16


BPB = 4


def _pool_body(x_ref, w_ref, b_ref, out_ref):
    bidx = pl.program_id(0)
    ones = jnp.ones((HW, 128), jnp.float32)
    for j in range(BPB):
        xb = x_ref[j].astype(jnp.bfloat16)
        logits = lax.dot_general(
            w_ref[...], xb, (((1,), (0,)), ((), ())),
            preferred_element_type=jnp.float32)
        ssum = lax.dot_general(
            logits, ones, (((1,), (0,)), ((), ())),
            precision=lax.Precision.HIGHEST,
            preferred_element_type=jnp.float32)
        out_ref[pl.ds(bidx * BPB + j, 1)] = (
            jnp.sum(ssum, axis=1) * SCALE + b_ref[0])[None]


def _lane_reduce(x, op):
    iota = lax.broadcasted_iota(jnp.int32, (L,), 0)
    dnums = lax.GatherDimensionNumbers(
        offset_dims=(), collapsed_slice_dims=(0,), start_index_map=(0,))
    for s in (8, 4, 2, 1):
        perm = jnp.reshape(iota ^ s, (L, 1))
        x = op(x, lax.gather(x, perm, dnums, (1,),
                             mode=lax.GatherScatterMode.PROMISE_IN_BOUNDS))
    return x


def _route_body(pooled_hbm, out_hbm, row_v, orow_v):
    wid = lax.axis_index("s") * NC + lax.axis_index("c")
    pltpu.sync_copy(pooled_hbm.at[wid], row_v)
    iota = lax.broadcasted_iota(jnp.int32, (L,), 0)
    v = [row_v[pl.ds(L * j, L)] for j in range(E // L)]
    gidx = [iota + L * j for j in range(E // L)]

    sel_vals = jnp.zeros((L,), jnp.float32)
    sel_idx = jnp.zeros((L,), jnp.int32)
    row_max = None
    for k in range(K):
        m = _lane_reduce(jnp.maximum(jnp.maximum(v[0], v[1]),
                                     jnp.maximum(v[2], v[3])), jnp.maximum)
        cand = [jnp.where(v[j] == m, gidx[j], E) for j in range(E // L)]
        idx = _lane_reduce(jnp.minimum(jnp.minimum(cand[0], cand[1]),
                                       jnp.minimum(cand[2], cand[3])),
                           jnp.minimum)
        sel_vals = jnp.where(iota == k, m, sel_vals)
        sel_idx = jnp.where(iota == k, idx, sel_idx)
        if row_max is None:
            row_max = m
        v = [jnp.where(gidx[j] == idx, jnp.float32(NEG), v[j])
             for j in range(E // L)]

    kmask = iota < K
    e = jnp.where(kmask, jnp.exp(sel_vals - row_max), 0.0)
    probs = e / _lane_reduce(e, jnp.add)

    zeros = jnp.zeros((L,), jnp.float32)
    for j in range(E // L):
        orow_v[pl.ds(L * j, L)] = zeros
    plsc.store_scatter(orow_v, [sel_idx], probs, mask=kmask)
    pltpu.sync_copy(orow_v, out_hbm.at[wid])


def kernel(x, W, b):
    xr = x.reshape(B, C, HW)
    Wb = W.astype(jnp.bfloat16)
    pooled = pl.pallas_call(
        _pool_body,
        grid=(B // BPB,),
        in_specs=[
            pl.BlockSpec((BPB, C, HW), lambda i: (i, 0, 0)),
            pl.BlockSpec((E, C), lambda i: (0, 0)),
            pl.BlockSpec((1, E), lambda i: (0, 0)),
        ],
        out_specs=pl.BlockSpec((B, E), lambda i: (0, 0)),
        out_shape=jax.ShapeDtypeStruct((B, E), jnp.float32),
        compiler_params=pltpu.CompilerParams(
            dimension_semantics=("arbitrary",)),
    )(xr, Wb, b.reshape(1, E))

    route = pl.kernel(
        _route_body,
        mesh=plsc.VectorSubcoreMesh(core_axis_name="c", subcore_axis_name="s"),
        out_type=jax.ShapeDtypeStruct((B, E), jnp.float32),
        scratch_types=[pltpu.VMEM((E,), jnp.float32),
                       pltpu.VMEM((E,), jnp.float32)],
        compiler_params=pltpu.CompilerParams(needs_layout_passes=False),
    )
    return route(pooled)
